# Initial kernel scaffold; baseline (speedup 1.0000x reference)
import jax
import jax.numpy as jnp
from jax.experimental import pallas as pl


def kernel(initial_ebs, l_rows_user, l_cols_user, l_vals_user, li_rows_user, li_cols_user, li_vals_user, l_rows_item, l_cols_item, l_vals_item, li_rows_item, li_cols_item, li_vals_item, W_side_l0_user, W_dot_l0_user, W_side_l0_item, W_dot_l0_item, W_side_l1_user, W_dot_l1_user, W_side_l1_item, W_dot_l1_item, cluster_no, train_flag):
    raise NotImplementedError("write your pallas kernel here")


# speedup vs baseline: 6.0712x; 6.0712x over previous
import functools

import jax
import jax.numpy as jnp
from jax import lax
from jax.experimental import pallas as pl
from jax.experimental.pallas import tpu as pltpu
from jax.experimental.pallas import tpu_sc as plsc

N = 10000
D = 128
NU = 5000
E = 160000
NC = 2
NS = 16
CHUNK = 128
CH = 79
EPT = CH * CHUNK
ROWS_PT = 320
ACC_ROWS = NS * ROWS_PT


def _prep_edges(rows, cols, vals):
    pad = NS * EPT - E
    rows = jnp.concatenate([rows, jnp.zeros((pad,), jnp.int32)])
    cols = jnp.concatenate([cols, jnp.zeros((pad,), jnp.int32)])
    vals = jnp.concatenate([vals, jnp.zeros((pad,), jnp.float32)])
    shp = (NS, CH, CHUNK)
    return rows.reshape(shp), cols.reshape(shp), vals.reshape(shp)


_sc_mesh = plsc.VectorSubcoreMesh(
    core_axis_name="c", subcore_axis_name="s", num_cores=NC, num_subcores=NS)


@functools.partial(
    pl.kernel,
    out_type=jax.ShapeDtypeStruct((NC, ACC_ROWS, D), jnp.float32),
    mesh=_sc_mesh,
    scratch_types=[
        pltpu.VMEM((2, CHUNK), jnp.int32),
        pltpu.VMEM((2, CHUNK), jnp.int32),
        pltpu.VMEM((CHUNK,), jnp.float32),
        pltpu.VMEM((CHUNK, D), jnp.float32),
        pltpu.VMEM_SHARED((ACC_ROWS, D), jnp.float32),
        pltpu.SemaphoreType.DMA,
    ],
)
def _sc_spmm(old_hbm, rows_hbm, cols_hbm, vals_hbm, zeros_hbm, out_hbm,
             colv, rowv, valv, gbuf, acc, sem):
    c = lax.axis_index("c")
    s = lax.axis_index("s")
    pltpu.sync_copy(zeros_hbm.at[pl.ds(s * ROWS_PT, ROWS_PT)],
                    acc.at[pl.ds(s * ROWS_PT, ROWS_PT)])
    plsc.subcore_barrier()

    def chunk_body(j, _):
        pltpu.sync_copy(cols_hbm.at[c, s, j], colv.at[0])
        pltpu.sync_copy(rows_hbm.at[c, s, j], rowv.at[0])
        pltpu.sync_copy(vals_hbm.at[c, s, j], valv)
        pltpu.async_copy(old_hbm.at[colv.at[0]], gbuf, sem).wait()

        def grp_body(g, _):
            vg = valv[pl.ds(g * 16, 16)]

            def row_body(j, _):
                i = g * 16 + j
                v16 = vg.at[jnp.full((16,), j, jnp.int32)].get(
                    mode="promise_in_bounds")
                for k in range(D // 16):
                    sl = pl.ds(k * 16, 16)
                    gbuf[i, sl] = gbuf[i, sl] * v16
                return 0

            lax.fori_loop(0, 16, row_body, 0)
            return 0

        lax.fori_loop(0, CHUNK // 16, grp_body, 0)
        pltpu.sync_copy(gbuf, acc.at[rowv.at[0]], add=True)
        return 0

    lax.fori_loop(0, CH, chunk_body, 0)
    plsc.subcore_barrier()
    pltpu.sync_copy(acc.at[pl.ds(s * ROWS_PT, ROWS_PT)],
                    out_hbm.at[c, pl.ds(s * ROWS_PT, ROWS_PT)])


BR = 1000


def _dense_body(acc_ref, old_ref, ws_ref, wd_ref, out_ref):
    p = acc_ref[0]
    o = old_ref[...]
    x = (jnp.dot(p + o, ws_ref[0], preferred_element_type=jnp.float32)
         + jnp.dot(p * o, wd_ref[0], preferred_element_type=jnp.float32))
    out_ref[...] = jnp.maximum(x, 0.2 * x)


def _tc_dense(acc2, old, ws_stack, wd_stack):
    return pl.pallas_call(
        _dense_body,
        grid=(2, NU // BR),
        in_specs=[
            pl.BlockSpec((1, BR, D), lambda e, b: (e, b, 0)),
            pl.BlockSpec((BR, D), lambda e, b: (e * (NU // BR) + b, 0)),
            pl.BlockSpec((1, D, D), lambda e, b: (e, 0, 0)),
            pl.BlockSpec((1, D, D), lambda e, b: (e, 0, 0)),
        ],
        out_specs=pl.BlockSpec((BR, D), lambda e, b: (e * (NU // BR) + b, 0)),
        out_shape=jax.ShapeDtypeStruct((N, D), jnp.float32),
    )(acc2, old, ws_stack, wd_stack)


def kernel(initial_ebs, l_rows_user, l_cols_user, l_vals_user,
           li_rows_user, li_cols_user, li_vals_user,
           l_rows_item, l_cols_item, l_vals_item,
           li_rows_item, li_cols_item, li_vals_item,
           W_side_l0_user, W_dot_l0_user, W_side_l0_item, W_dot_l0_item,
           W_side_l1_user, W_dot_l1_user, W_side_l1_item, W_dot_l1_item,
           cluster_no, train_flag):
    ru, cu, vu = _prep_edges(l_rows_user, l_cols_user, l_vals_user)
    ri, ci, vi = _prep_edges(l_rows_item, l_cols_item, l_vals_item)
    rows_all = jnp.stack([ru, ri])
    cols_all = jnp.stack([cu, ci])
    vals_all = jnp.stack([vu, vi])
    zeros = jnp.zeros((ACC_ROWS, D), jnp.float32)

    ws = [jnp.stack([W_side_l0_user, W_side_l0_item]),
          jnp.stack([W_side_l1_user, W_side_l1_item])]
    wd = [jnp.stack([W_dot_l0_user, W_dot_l0_item]),
          jnp.stack([W_dot_l1_user, W_dot_l1_item])]

    old = initial_ebs
    outs = []
    for l in range(2):
        acc2 = _sc_spmm(old, rows_all, cols_all, vals_all, zeros)
        old = _tc_dense(acc2, old, ws[l], wd[l])
        outs.append(old)
    return jnp.concatenate(outs, axis=0)
